# Initial kernel scaffold; baseline (speedup 1.0000x reference)
import jax
import jax.numpy as jnp
from jax.experimental import pallas as pl


def kernel(x, edge_index, batch, W1, b1, W2, b2, W3, b3, Wc1, bc1, Wc2, bc2):
    raise NotImplementedError("write your pallas kernel here")


# speedup vs baseline: 14.7564x; 14.7564x over previous
import functools

import jax
import jax.numpy as jnp
from jax import lax
from jax.experimental import pallas as pl
from jax.experimental.pallas import tpu as pltpu
from jax.experimental.pallas import tpu_sc as plsc

N = 10000
E = 320000
F_IN = 128
H = 64
H2 = 32
C = 12
B = 64

NC = 2
NS = 16
NW = NC * NS
CK = 128
CPT = 80
EP = NW * CPT * CK
NP = 10240

_f32 = jnp.float32


def _sc_mesh():
    return plsc.VectorSubcoreMesh(core_axis_name="c", subcore_axis_name="s")


_SC_PARAMS = pltpu.CompilerParams(use_tc_tiling_on_sc=False)


def _make_deg_kernel():
    rpt = NP // NS

    @functools.partial(
        pl.kernel,
        out_type=jax.ShapeDtypeStruct((NC, NP, 16), _f32),
        mesh=_sc_mesh(),
        compiler_params=_SC_PARAMS,
        scratch_types=[
            pltpu.VMEM((CPT, CK), jnp.int32),
            pltpu.VMEM((CK, 16), _f32),
            pltpu.VMEM_SHARED((NP, 16), _f32),
        ],
    )
    def deg_kernel(dstp_hbm, zeros_hbm, ones_hbm, out_hbm, dst_v, ones_v, acc_sh):
        cid = lax.axis_index("c")
        sid = lax.axis_index("s")
        wid = sid * NC + cid
        pltpu.sync_copy(dstp_hbm.at[pl.ds(wid * CPT, CPT)], dst_v)
        pltpu.sync_copy(ones_hbm, ones_v)
        pltpu.sync_copy(zeros_hbm.at[pl.ds(sid * rpt, rpt)],
                        acc_sh.at[pl.ds(sid * rpt, rpt)])
        plsc.subcore_barrier()

        def body(j, carry):
            pltpu.sync_copy(ones_v, acc_sh.at[dst_v.at[j]], add=True)
            return carry

        lax.fori_loop(0, CPT, body, 0)
        plsc.subcore_barrier()
        pltpu.sync_copy(acc_sh.at[pl.ds(sid * rpt, rpt)],
                        out_hbm.at[cid, pl.ds(sid * rpt, rpt)])

    return deg_kernel


def _make_prop_kernel(hc):
    rpt = NP // NS

    @functools.partial(
        pl.kernel,
        out_type=jax.ShapeDtypeStruct((NC, NP, hc), _f32),
        mesh=_sc_mesh(),
        compiler_params=_SC_PARAMS,
        scratch_types=[
            pltpu.VMEM((CPT, CK), jnp.int32),
            pltpu.VMEM((CPT, CK), jnp.int32),
            pltpu.VMEM((2, CK, hc), _f32),
            pltpu.VMEM_SHARED((NP, hc), _f32),
            pltpu.SemaphoreType.DMA,
        ],
    )
    def prop_kernel(g_hbm, srcp_hbm, dstp_hbm, zeros_hbm, out_hbm,
                    src_v, dst_v, rows_v, acc_sh, sem):
        cid = lax.axis_index("c")
        sid = lax.axis_index("s")
        wid = sid * NC + cid
        pltpu.sync_copy(srcp_hbm.at[pl.ds(wid * CPT, CPT)], src_v)
        pltpu.sync_copy(dstp_hbm.at[pl.ds(wid * CPT, CPT)], dst_v)
        pltpu.sync_copy(zeros_hbm.at[pl.ds(sid * rpt, rpt)],
                        acc_sh.at[pl.ds(sid * rpt, rpt)])
        plsc.subcore_barrier()

        pltpu.async_copy(g_hbm.at[src_v.at[0]], rows_v.at[0], sem)

        def body(j, carry):
            for b in range(2):
                jj = j + b
                pltpu.make_async_copy(g_hbm.at[src_v.at[jj]],
                                      rows_v.at[b], sem).wait()

                @pl.when(jj + 1 < CPT)
                def _():
                    pltpu.async_copy(g_hbm.at[src_v.at[jj + 1]],
                                     rows_v.at[1 - b], sem)

                pltpu.sync_copy(rows_v.at[b], acc_sh.at[dst_v.at[jj]], add=True)
            return carry

        lax.fori_loop(0, CPT // 2, lambda j, c: body(2 * j, c), 0)
        plsc.subcore_barrier()
        pltpu.sync_copy(acc_sh.at[pl.ds(sid * rpt, rpt)],
                        out_hbm.at[cid, pl.ds(sid * rpt, rpt)])

    return prop_kernel


_RB = 512


def _dot(a, b):
    return jax.lax.dot_general(a, b, (((1,), (0,)), ((), ())),
                               preferred_element_type=_f32,
                               precision=jax.lax.Precision.HIGHEST)


def _tc_prelude(deg_parts, xpad, W1):
    grid = NP // _RB

    def body(deg_ref, x_ref, w_ref, g_ref, dinv_ref, dinvm_ref):
        i = pl.program_id(0)
        deg = deg_ref[0, :, :1] + deg_ref[1, :, :1] + 1.0
        dinv = lax.rsqrt(deg)
        ridx = lax.broadcasted_iota(jnp.int32, (_RB, 1), 0) + i * _RB
        dinvm = jnp.where(ridx < N, dinv, 0.0)
        g_ref[...] = _dot(x_ref[...], w_ref[...]) * dinvm
        dinv_ref[...] = jnp.broadcast_to(dinv, (_RB, 8))
        dinvm_ref[...] = jnp.broadcast_to(dinvm, (_RB, 8))

    return pl.pallas_call(
        body,
        grid=(grid,),
        in_specs=[
            pl.BlockSpec((NC, _RB, 16), lambda i: (0, i, 0)),
            pl.BlockSpec((_RB, F_IN), lambda i: (i, 0)),
            pl.BlockSpec((F_IN, H), lambda i: (0, 0)),
        ],
        out_specs=[
            pl.BlockSpec((_RB, H), lambda i: (i, 0)),
            pl.BlockSpec((_RB, 8), lambda i: (i, 0)),
            pl.BlockSpec((_RB, 8), lambda i: (i, 0)),
        ],
        out_shape=[
            jax.ShapeDtypeStruct((NP, H), _f32),
            jax.ShapeDtypeStruct((NP, 8), _f32),
            jax.ShapeDtypeStruct((NP, 8), _f32),
        ],
    )(deg_parts, xpad, W1)


def _tc_layer(parts, g_prev, dinv8, dinvm8, b_prev, W_next, h_in, h_out):
    grid = NP // _RB

    def body(p_ref, g_ref, dinv_ref, dinvm_ref, b_ref, w_ref, out_ref):
        comb = (p_ref[0] + p_ref[1] + g_ref[...]) * dinv_ref[:, :1] + b_ref[...]
        h = jnp.maximum(comb, 0.0)
        out_ref[...] = _dot(h, w_ref[...]) * dinvm_ref[:, :1]

    return pl.pallas_call(
        body,
        grid=(grid,),
        in_specs=[
            pl.BlockSpec((NC, _RB, h_in), lambda i: (0, i, 0)),
            pl.BlockSpec((_RB, h_in), lambda i: (i, 0)),
            pl.BlockSpec((_RB, 8), lambda i: (i, 0)),
            pl.BlockSpec((_RB, 8), lambda i: (i, 0)),
            pl.BlockSpec((1, h_in), lambda i: (0, 0)),
            pl.BlockSpec((h_in, h_out), lambda i: (0, 0)),
        ],
        out_specs=pl.BlockSpec((_RB, h_out), lambda i: (i, 0)),
        out_shape=jax.ShapeDtypeStruct((NP, h_out), _f32),
    )(parts, g_prev, dinv8, dinvm8, b_prev, W_next)


_RD = 1000


def _tc_head(parts, g_prev, dinv8, b_prev, batch2d, Wc1, bc1, Wc2, bc2):
    grid = N // _RD

    def body(p_ref, g_ref, dinv_ref, b_ref, bat_ref, wc1_ref, bc1_ref,
             wc2_ref, bc2_ref, out_ref, pooled, counts):
        i = pl.program_id(0)

        @pl.when(i == 0)
        def _():
            pooled[...] = jnp.zeros((B, H2), _f32)
            counts[...] = jnp.zeros((B, 8), _f32)

        comb = (p_ref[0] + p_ref[1] + g_ref[...]) * dinv_ref[:, :1] + b_ref[...]
        h = jnp.maximum(comb, 0.0)
        iota_b = lax.broadcasted_iota(jnp.int32, (1, B), 1)
        onehot = (bat_ref[...] == iota_b).astype(_f32)
        pooled[...] += jax.lax.dot_general(
            onehot, h, (((0,), (0,)), ((), ())),
            preferred_element_type=_f32,
            precision=jax.lax.Precision.HIGHEST)
        counts[...] += jax.lax.dot_general(
            onehot, jnp.ones((_RD, 8), _f32), (((0,), (0,)), ((), ())),
            preferred_element_type=_f32,
            precision=jax.lax.Precision.HIGHEST)

        @pl.when(i == grid - 1)
        def _():
            mean = pooled[...] / jnp.maximum(counts[:, :1], 1.0)
            z = jnp.maximum(_dot(mean, wc1_ref[...]) + bc1_ref[...], 0.0)
            logits = _dot(z, wc2_ref[...]) + bc2_ref[...]
            m = jnp.max(logits, axis=-1, keepdims=True)
            ex = jnp.exp(logits - m)
            out_ref[...] = (logits - m) - jnp.log(jnp.sum(ex, -1, keepdims=True))

    return pl.pallas_call(
        body,
        grid=(grid,),
        in_specs=[
            pl.BlockSpec((NC, _RD, H2), lambda i: (0, i, 0)),
            pl.BlockSpec((_RD, H2), lambda i: (i, 0)),
            pl.BlockSpec((_RD, 8), lambda i: (i, 0)),
            pl.BlockSpec((1, H2), lambda i: (0, 0)),
            pl.BlockSpec((_RD, 1), lambda i: (i, 0)),
            pl.BlockSpec((H2, 16), lambda i: (0, 0)),
            pl.BlockSpec((1, 16), lambda i: (0, 0)),
            pl.BlockSpec((16, C), lambda i: (0, 0)),
            pl.BlockSpec((1, C), lambda i: (0, 0)),
        ],
        out_specs=pl.BlockSpec((B, C), lambda i: (0, 0)),
        out_shape=jax.ShapeDtypeStruct((B, C), _f32),
        scratch_shapes=[
            pltpu.VMEM((B, H2), _f32),
            pltpu.VMEM((B, 8), _f32),
        ],
    )(parts, g_prev, dinv8, b_prev, batch2d, Wc1, bc1, Wc2, bc2)


_deg_kernel = _make_deg_kernel()
_prop64 = _make_prop_kernel(H)
_prop32 = _make_prop_kernel(H2)


def kernel(x, edge_index, batch, W1, b1, W2, b2, W3, b3, Wc1, bc1, Wc2, bc2):
    src = edge_index[0].astype(jnp.int32)
    dst = edge_index[1].astype(jnp.int32)
    padi = jnp.full((EP - E,), N, jnp.int32)
    srcp = jnp.concatenate([src, padi]).reshape(NW * CPT, CK)
    dstp = jnp.concatenate([dst, padi]).reshape(NW * CPT, CK)
    xpad = jnp.zeros((NP, F_IN), _f32).at[:N].set(x)
    zeros16 = jnp.zeros((NP, 16), _f32)
    ones16 = jnp.ones((CK, 16), _f32)
    zeros64 = jnp.zeros((NP, H), _f32)
    zeros32 = jnp.zeros((NP, H2), _f32)

    deg_parts = _deg_kernel(dstp, zeros16, ones16)
    g1, dinv8, dinvm8 = _tc_prelude(deg_parts, xpad, W1)
    p1 = _prop64(g1, srcp, dstp, zeros64)
    g2 = _tc_layer(p1, g1, dinv8, dinvm8, b1.reshape(1, H), W2, H, H)
    p2 = _prop64(g2, srcp, dstp, zeros64)
    g3 = _tc_layer(p2, g2, dinv8, dinvm8, b2.reshape(1, H), W3, H, H2)
    p3 = _prop32(g3, srcp, dstp, zeros32)
    return _tc_head(p3, g3, dinv8, b3.reshape(1, H2),
                    batch.reshape(N, 1).astype(jnp.int32),
                    Wc1, bc1.reshape(1, 16), Wc2, bc2.reshape(1, C))


# speedup vs baseline: 30.3961x; 2.0599x over previous
import functools

import jax
import jax.numpy as jnp
from jax import lax
from jax.experimental import pallas as pl
from jax.experimental.pallas import tpu as pltpu
from jax.experimental.pallas import tpu_sc as plsc

N = 10000
E = 320000
F_IN = 128
H = 64
H2 = 32
C = 12
B = 64

NC = 2
NS = 16
NW = NC * NS
CK = 128
CPT = 80
EP = NW * CPT * CK
NP = 10240

_f32 = jnp.float32


def _sc_mesh():
    return plsc.VectorSubcoreMesh(core_axis_name="c", subcore_axis_name="s")


_SC_PARAMS = pltpu.CompilerParams(use_tc_tiling_on_sc=False)


def _make_deg_kernel():
    rpt = NP // NS

    @functools.partial(
        pl.kernel,
        out_type=jax.ShapeDtypeStruct((NC, NP, 16), _f32),
        mesh=_sc_mesh(),
        compiler_params=_SC_PARAMS,
        scratch_types=[
            pltpu.VMEM((CPT, CK), jnp.int32),
            pltpu.VMEM((CK, 16), _f32),
            pltpu.VMEM_SHARED((NP, 16), _f32),
        ],
    )
    def deg_kernel(dstp_hbm, zeros_hbm, ones_hbm, out_hbm, dst_v, ones_v, acc_sh):
        cid = lax.axis_index("c")
        sid = lax.axis_index("s")
        wid = sid * NC + cid
        pltpu.sync_copy(dstp_hbm.at[pl.ds(wid * CPT, CPT)], dst_v)
        pltpu.sync_copy(ones_hbm, ones_v)
        pltpu.sync_copy(zeros_hbm.at[pl.ds(sid * rpt, rpt)],
                        acc_sh.at[pl.ds(sid * rpt, rpt)])
        plsc.subcore_barrier()

        def body(j, carry):
            pltpu.sync_copy(ones_v, acc_sh.at[dst_v.at[j]], add=True)
            return carry

        lax.fori_loop(0, CPT, body, 0)
        plsc.subcore_barrier()
        pltpu.sync_copy(acc_sh.at[pl.ds(sid * rpt, rpt)],
                        out_hbm.at[cid, pl.ds(sid * rpt, rpt)])

    return deg_kernel


def _make_prop_kernel(hc):
    rpt = NP // NS

    @functools.partial(
        pl.kernel,
        out_type=jax.ShapeDtypeStruct((NC, NP, hc), _f32),
        mesh=_sc_mesh(),
        compiler_params=_SC_PARAMS,
        scratch_types=[
            pltpu.VMEM((CPT, CK), jnp.int32),
            pltpu.VMEM((CPT, CK), jnp.int32),
            pltpu.VMEM((2, CK, hc), _f32),
            pltpu.VMEM_SHARED((NP, hc), _f32),
            pltpu.VMEM_SHARED((NP, hc), _f32),
            pltpu.SemaphoreType.DMA,
        ],
    )
    def prop_kernel(g_hbm, srcp_hbm, dstp_hbm, zeros_hbm, out_hbm,
                    src_v, dst_v, rows_v, acc_sh, g_sh, sem):
        cid = lax.axis_index("c")
        sid = lax.axis_index("s")
        wid = sid * NC + cid
        pltpu.sync_copy(srcp_hbm.at[pl.ds(wid * CPT, CPT)], src_v)
        pltpu.sync_copy(dstp_hbm.at[pl.ds(wid * CPT, CPT)], dst_v)
        pltpu.sync_copy(g_hbm.at[pl.ds(sid * rpt, rpt)],
                        g_sh.at[pl.ds(sid * rpt, rpt)])
        pltpu.sync_copy(zeros_hbm.at[pl.ds(sid * rpt, rpt)],
                        acc_sh.at[pl.ds(sid * rpt, rpt)])
        plsc.subcore_barrier()

        pltpu.async_copy(g_sh.at[src_v.at[0]], rows_v.at[0], sem)

        def body(j, carry):
            for b in range(2):
                jj = j + b
                pltpu.make_async_copy(g_sh.at[src_v.at[jj]],
                                      rows_v.at[b], sem).wait()

                @pl.when(jj + 1 < CPT)
                def _():
                    pltpu.async_copy(g_sh.at[src_v.at[jj + 1]],
                                     rows_v.at[1 - b], sem)

                pltpu.sync_copy(rows_v.at[b], acc_sh.at[dst_v.at[jj]], add=True)
            return carry

        lax.fori_loop(0, CPT // 2, lambda j, c: body(2 * j, c), 0)
        plsc.subcore_barrier()
        pltpu.sync_copy(acc_sh.at[pl.ds(sid * rpt, rpt)],
                        out_hbm.at[cid, pl.ds(sid * rpt, rpt)])

    return prop_kernel


_RB = 512


def _dot(a, b):
    return jax.lax.dot_general(a, b, (((1,), (0,)), ((), ())),
                               preferred_element_type=_f32,
                               precision=jax.lax.Precision.HIGHEST)


def _tc_prelude(deg_parts, xpad, W1):
    grid = NP // _RB

    def body(deg_ref, x_ref, w_ref, g_ref, dinv_ref, dinvm_ref):
        i = pl.program_id(0)
        deg = deg_ref[0, :, :1] + deg_ref[1, :, :1] + 1.0
        dinv = lax.rsqrt(deg)
        ridx = lax.broadcasted_iota(jnp.int32, (_RB, 1), 0) + i * _RB
        dinvm = jnp.where(ridx < N, dinv, 0.0)
        g_ref[...] = _dot(x_ref[...], w_ref[...]) * dinvm
        dinv_ref[...] = jnp.broadcast_to(dinv, (_RB, 8))
        dinvm_ref[...] = jnp.broadcast_to(dinvm, (_RB, 8))

    return pl.pallas_call(
        body,
        grid=(grid,),
        in_specs=[
            pl.BlockSpec((NC, _RB, 16), lambda i: (0, i, 0)),
            pl.BlockSpec((_RB, F_IN), lambda i: (i, 0)),
            pl.BlockSpec((F_IN, H), lambda i: (0, 0)),
        ],
        out_specs=[
            pl.BlockSpec((_RB, H), lambda i: (i, 0)),
            pl.BlockSpec((_RB, 8), lambda i: (i, 0)),
            pl.BlockSpec((_RB, 8), lambda i: (i, 0)),
        ],
        out_shape=[
            jax.ShapeDtypeStruct((NP, H), _f32),
            jax.ShapeDtypeStruct((NP, 8), _f32),
            jax.ShapeDtypeStruct((NP, 8), _f32),
        ],
    )(deg_parts, xpad, W1)


def _tc_layer(parts, g_prev, dinv8, dinvm8, b_prev, W_next, h_in, h_out):
    grid = NP // _RB

    def body(p_ref, g_ref, dinv_ref, dinvm_ref, b_ref, w_ref, out_ref):
        comb = (p_ref[0] + p_ref[1] + g_ref[...]) * dinv_ref[:, :1] + b_ref[...]
        h = jnp.maximum(comb, 0.0)
        out_ref[...] = _dot(h, w_ref[...]) * dinvm_ref[:, :1]

    return pl.pallas_call(
        body,
        grid=(grid,),
        in_specs=[
            pl.BlockSpec((NC, _RB, h_in), lambda i: (0, i, 0)),
            pl.BlockSpec((_RB, h_in), lambda i: (i, 0)),
            pl.BlockSpec((_RB, 8), lambda i: (i, 0)),
            pl.BlockSpec((_RB, 8), lambda i: (i, 0)),
            pl.BlockSpec((1, h_in), lambda i: (0, 0)),
            pl.BlockSpec((h_in, h_out), lambda i: (0, 0)),
        ],
        out_specs=pl.BlockSpec((_RB, h_out), lambda i: (i, 0)),
        out_shape=jax.ShapeDtypeStruct((NP, h_out), _f32),
    )(parts, g_prev, dinv8, dinvm8, b_prev, W_next)


_RD = 1000


def _tc_head(parts, g_prev, dinv8, b_prev, batch2d, Wc1, bc1, Wc2, bc2):
    grid = N // _RD

    def body(p_ref, g_ref, dinv_ref, b_ref, bat_ref, wc1_ref, bc1_ref,
             wc2_ref, bc2_ref, out_ref, pooled, counts):
        i = pl.program_id(0)

        @pl.when(i == 0)
        def _():
            pooled[...] = jnp.zeros((B, H2), _f32)
            counts[...] = jnp.zeros((B, 8), _f32)

        comb = (p_ref[0] + p_ref[1] + g_ref[...]) * dinv_ref[:, :1] + b_ref[...]
        h = jnp.maximum(comb, 0.0)
        iota_b = lax.broadcasted_iota(jnp.int32, (1, B), 1)
        onehot = (bat_ref[...] == iota_b).astype(_f32)
        pooled[...] += jax.lax.dot_general(
            onehot, h, (((0,), (0,)), ((), ())),
            preferred_element_type=_f32,
            precision=jax.lax.Precision.HIGHEST)
        counts[...] += jax.lax.dot_general(
            onehot, jnp.ones((_RD, 8), _f32), (((0,), (0,)), ((), ())),
            preferred_element_type=_f32,
            precision=jax.lax.Precision.HIGHEST)

        @pl.when(i == grid - 1)
        def _():
            mean = pooled[...] / jnp.maximum(counts[:, :1], 1.0)
            z = jnp.maximum(_dot(mean, wc1_ref[...]) + bc1_ref[...], 0.0)
            logits = _dot(z, wc2_ref[...]) + bc2_ref[...]
            m = jnp.max(logits, axis=-1, keepdims=True)
            ex = jnp.exp(logits - m)
            out_ref[...] = (logits - m) - jnp.log(jnp.sum(ex, -1, keepdims=True))

    return pl.pallas_call(
        body,
        grid=(grid,),
        in_specs=[
            pl.BlockSpec((NC, _RD, H2), lambda i: (0, i, 0)),
            pl.BlockSpec((_RD, H2), lambda i: (i, 0)),
            pl.BlockSpec((_RD, 8), lambda i: (i, 0)),
            pl.BlockSpec((1, H2), lambda i: (0, 0)),
            pl.BlockSpec((_RD, 1), lambda i: (i, 0)),
            pl.BlockSpec((H2, 16), lambda i: (0, 0)),
            pl.BlockSpec((1, 16), lambda i: (0, 0)),
            pl.BlockSpec((16, C), lambda i: (0, 0)),
            pl.BlockSpec((1, C), lambda i: (0, 0)),
        ],
        out_specs=pl.BlockSpec((B, C), lambda i: (0, 0)),
        out_shape=jax.ShapeDtypeStruct((B, C), _f32),
        scratch_shapes=[
            pltpu.VMEM((B, H2), _f32),
            pltpu.VMEM((B, 8), _f32),
        ],
    )(parts, g_prev, dinv8, b_prev, batch2d, Wc1, bc1, Wc2, bc2)


_deg_kernel = _make_deg_kernel()
_prop64 = _make_prop_kernel(H)
_prop32 = _make_prop_kernel(H2)


def kernel(x, edge_index, batch, W1, b1, W2, b2, W3, b3, Wc1, bc1, Wc2, bc2):
    src = edge_index[0].astype(jnp.int32)
    dst = edge_index[1].astype(jnp.int32)
    padi = jnp.full((EP - E,), N, jnp.int32)
    srcp = jnp.concatenate([src, padi]).reshape(NW * CPT, CK)
    dstp = jnp.concatenate([dst, padi]).reshape(NW * CPT, CK)
    xpad = jnp.zeros((NP, F_IN), _f32).at[:N].set(x)
    zeros16 = jnp.zeros((NP, 16), _f32)
    ones16 = jnp.ones((CK, 16), _f32)
    zeros64 = jnp.zeros((NP, H), _f32)
    zeros32 = jnp.zeros((NP, H2), _f32)

    deg_parts = _deg_kernel(dstp, zeros16, ones16)
    g1, dinv8, dinvm8 = _tc_prelude(deg_parts, xpad, W1)
    p1 = _prop64(g1, srcp, dstp, zeros64)
    g2 = _tc_layer(p1, g1, dinv8, dinvm8, b1.reshape(1, H), W2, H, H)
    p2 = _prop64(g2, srcp, dstp, zeros64)
    g3 = _tc_layer(p2, g2, dinv8, dinvm8, b2.reshape(1, H), W3, H, H2)
    p3 = _prop32(g3, srcp, dstp, zeros32)
    return _tc_head(p3, g3, dinv8, b3.reshape(1, H2),
                    batch.reshape(N, 1).astype(jnp.int32),
                    Wc1, bc1.reshape(1, 16), Wc2, bc2.reshape(1, C))
